# baseline (speedup 1.0000x reference)
import functools

import jax
import jax.numpy as jnp
from jax.experimental import pallas as pl

T = 16384
D = 2048
H = 1024
E = 16
K = 2
TM = 512


def _router_kernel(x_ref, w1_ref, b1_ref, w2_ref, b2_ref,
                   w_ref, i_ref, p_ref):
    x = x_ref[...]
    h = jnp.dot(x, w1_ref[...], preferred_element_type=jnp.float32)
    h = jnp.maximum(h + b1_ref[...], 0.0)
    logits = jnp.dot(h, w2_ref[...], preferred_element_type=jnp.float32)
    logits = logits + b2_ref[...]

    m = jnp.max(logits, axis=1, keepdims=True)
    e = jnp.exp(logits - m)
    probs = e / jnp.sum(e, axis=1, keepdims=True)

    iota = jax.lax.broadcasted_iota(jnp.int32, (TM, E), 1)
    m1 = jnp.max(probs, axis=1, keepdims=True)
    i1 = jnp.min(jnp.where(probs == m1, iota, E), axis=1, keepdims=True)
    masked = jnp.where(iota == i1, -1.0, probs)
    m2 = jnp.max(masked, axis=1, keepdims=True)
    i2 = jnp.min(jnp.where(masked == m2, iota, E), axis=1, keepdims=True)

    denom = jnp.maximum(m1 + m2, 1e-6)
    w_ref[...] = jnp.concatenate([m1, m2], axis=1) / denom
    i_ref[...] = jnp.concatenate([i1, i2], axis=1)
    p_ref[...] = probs


@functools.partial(jax.jit, static_argnames=("interpret",))
def kernel(pooled_feat, W1, b1, W2, b2, interpret=False):
    b1r = b1.reshape(1, H)
    b2r = b2.reshape(1, E)
    grid = (T // TM,)
    out = pl.pallas_call(
        _router_kernel,
        grid=grid,
        in_specs=[
            pl.BlockSpec((TM, D), lambda i: (i, 0)),
            pl.BlockSpec((D, H), lambda i: (0, 0)),
            pl.BlockSpec((1, H), lambda i: (0, 0)),
            pl.BlockSpec((H, E), lambda i: (0, 0)),
            pl.BlockSpec((1, E), lambda i: (0, 0)),
        ],
        out_specs=[
            pl.BlockSpec((TM, K), lambda i: (i, 0)),
            pl.BlockSpec((TM, K), lambda i: (i, 0)),
            pl.BlockSpec((TM, E), lambda i: (i, 0)),
        ],
        out_shape=[
            jax.ShapeDtypeStruct((T, K), jnp.float32),
            jax.ShapeDtypeStruct((T, K), jnp.int32),
            jax.ShapeDtypeStruct((T, E), jnp.float32),
        ],
        interpret=interpret,
    )(pooled_feat, W1, b1r, W2, b2r)
    return (out[0], out[1], out[2])
